# baseline (speedup 1.0000x reference)
import functools

import jax
import jax.numpy as jnp
from jax import lax
from jax.experimental import pallas as pl
from jax.experimental.pallas import tpu as pltpu
from jax.experimental.pallas import tpu_sc as plsc

_HYP, _SPH, _EUC = 16, 16, 16
_TOTAL = _HYP + _SPH + _EUC
_MAX_NORM = 1.0 - 1e-5

_NC, _NS = 2, 16
_NW = _NC * _NS


_NBUF = 16


def _sc_gather_selected(tableT, codes, b):
  pairs_per_w = b // _NW
  dmas_per_w = 2 * pairs_per_w
  groups = dmas_per_w // _NBUF

  mesh = plsc.VectorSubcoreMesh(core_axis_name="c", subcore_axis_name="s")

  @functools.partial(
      pl.kernel,
      out_type=jax.ShapeDtypeStruct((b, 32), jnp.float32),
      mesh=mesh,
      compiler_params=pltpu.CompilerParams(
          use_tc_tiling_on_sc=True, needs_layout_passes=False
      ),
      scratch_types=[
          pltpu.VMEM((dmas_per_w,), jnp.int32),
          pltpu.VMEM((_NBUF, 16, 128), jnp.float32),
          pltpu.VMEM((pairs_per_w, 32), jnp.float32),
          pltpu.SemaphoreType.DMA,
          pltpu.SemaphoreType.DMA((_NBUF,)),
      ],
  )
  def gather_kernel(tableT_hbm, codes_hbm, out_hbm, codes_v, slots, rows_v,
                    sem_c, sems):
    wid = lax.axis_index("s") * _NC + lax.axis_index("c")
    base = wid * pairs_per_w
    pltpu.async_copy(
        codes_hbm.at[pl.ds(wid * dmas_per_w, dmas_per_w)], codes_v, sem_c
    ).wait()

    iota16 = lax.iota(jnp.int32, 16)

    def lane_scalar(vec, b):
      return jnp.sum(jnp.where(iota16 == b, vec, 0))

    def fire(code, slot):
      ent = code >> 2
      off = (code & 3) << 4
      eb = pl.multiple_of((ent >> 7) << 7, 128)
      pltpu.async_copy(
          tableT_hbm.at[pl.ds(pl.multiple_of(off, 8), 8), pl.ds(eb, 128)],
          slots.at[slot, pl.ds(0, 8)],
          sems.at[slot],
      )
      pltpu.async_copy(
          tableT_hbm.at[pl.ds(pl.multiple_of(off + 8, 8), 8), pl.ds(eb, 128)],
          slots.at[slot, pl.ds(8, 8)],
          sems.at[slot],
      )

    def extract(code, j, slot):
      pltpu.make_async_copy(
          tableT_hbm.at[pl.ds(0, 16), pl.ds(0, 128)],
          slots.at[slot],
          sems.at[slot],
      ).wait()
      lane = jnp.full((16,), (code >> 2) & 127, jnp.int32)
      vals = plsc.load_gather(slots.at[slot], [iota16, lane])
      rows_v[j >> 1, pl.ds((j & 1) * 16, 16)] = vals

    cv0 = codes_v[pl.ds(0, _NBUF)]
    for bslot in range(_NBUF):
      fire(lane_scalar(cv0, bslot), bslot)

    def body(g, carry):
      jb = g * _NBUF
      cv_cur = codes_v[pl.ds(jb, _NBUF)]
      cv_nxt = codes_v[pl.ds(jb + _NBUF, _NBUF)]
      for bslot in range(_NBUF):
        extract(lane_scalar(cv_cur, bslot), jb + bslot, bslot)
        fire(lane_scalar(cv_nxt, bslot), bslot)
      return carry

    lax.fori_loop(0, groups - 1, body, 0)
    jb_last = (groups - 1) * _NBUF
    cv_last = codes_v[pl.ds(jb_last, _NBUF)]
    for bslot in range(_NBUF):
      extract(lane_scalar(cv_last, bslot), jb_last + bslot, bslot)

    pltpu.sync_copy(rows_v, out_hbm.at[pl.ds(base, pairs_per_w)])

  return gather_kernel(tableT, codes)


def _tc_body(x_ref, wh_ref, bh_ref, ws_ref, bs_ref, we_ref, be_ref,
             rel_ref, out_ref):
  x = x_ref[...]
  h = x[:16, :]
  t = x[16:, :]

  def hyp_project(v):
    n = jnp.sqrt(jnp.sum(v * v, axis=0, keepdims=True))
    f = jnp.minimum(1.0, _MAX_NORM / jnp.maximum(n, 1e-15))
    return v * f

  def sph_project(v):
    n = jnp.sqrt(jnp.sum(v * v, axis=0, keepdims=True))
    return v / jnp.maximum(n, 1e-7)

  hh = hyp_project(h)
  th = hyp_project(t)
  q = jnp.dot(wh_ref[...], hh, preferred_element_type=jnp.float32)
  q = q + bh_ref[...]
  xh = hyp_project(q)
  sq = jnp.sum((xh - th) ** 2, axis=0, keepdims=True)
  xn = jnp.sum(xh * xh, axis=0, keepdims=True)
  yn = jnp.sum(th * th, axis=0, keepdims=True)
  arg = 1.0 + 2.0 * sq / jnp.maximum((1.0 - xn) * (1.0 - yn), 1e-15)
  arg = jnp.maximum(arg, 1.0 + 1e-7)
  d_hyp = jnp.log(arg + jnp.sqrt((arg - 1.0) * (arg + 1.0)))

  hs = sph_project(h)
  ts = sph_project(t)
  s = jnp.dot(ws_ref[...], hs, preferred_element_type=jnp.float32)
  s = s + bs_ref[...]
  sn = jnp.sqrt(jnp.sum(s * s, axis=0, keepdims=True))
  shat = s / jnp.maximum(sn, 1e-7)
  dd = jnp.sum(shat * ts, axis=0, keepdims=True)
  dd = jnp.clip(dd, -1.0 + 1e-7, 1.0 - 1e-7)
  d_sph = jnp.arctan2(jnp.sqrt(1.0 - dd * dd), dd)

  e = jnp.dot(we_ref[...], h, preferred_element_type=jnp.float32)
  e = e + be_ref[...]
  d_euc = jnp.sqrt(jnp.sum((e - t) ** 2, axis=0, keepdims=True))

  rel = rel_ref[...]
  dist = jnp.where(rel == 0, d_hyp, jnp.where(rel == 1, d_sph, d_euc))
  out_ref[...] = -dist


def _tc_compute(pairsT, wh, bh, ws, bs, we, be, relT):
  b = relT.shape[1]
  blk = 2048
  grid = (b // blk,)
  small = lambda shape: pl.BlockSpec(shape, lambda i: (0,) * len(shape))
  return pl.pallas_call(
      _tc_body,
      grid=grid,
      in_specs=[
          pl.BlockSpec((32, blk), lambda i: (0, i)),
          small((_HYP, _HYP)),
          small((_HYP, 1)),
          small((_SPH, _SPH)),
          small((_SPH, 1)),
          small((_EUC, _EUC)),
          small((_EUC, 1)),
          pl.BlockSpec((1, blk), lambda i: (0, i)),
      ],
      out_specs=pl.BlockSpec((1, blk), lambda i: (0, i)),
      out_shape=jax.ShapeDtypeStruct((1, b), jnp.float32),
  )(pairsT, wh, bh, ws, bs, we, be, relT)


def kernel(entity_embeddings, W_h, b_h, W_s, b_s, W_e, b_e, heads, relations,
           tails):
  b = heads.shape[0]
  rel = relations.astype(jnp.int32)
  hc = (heads.astype(jnp.int32) << 2) | rel
  tc = (tails.astype(jnp.int32) << 2) | rel
  codes = jnp.stack([hc, tc], axis=1).reshape(2 * b)
  tableT = entity_embeddings.T

  pairs = _sc_gather_selected(tableT, codes, b)

  out = _tc_compute(
      pairs.T,
      W_h, b_h.reshape(_HYP, 1),
      W_s, b_s.reshape(_SPH, 1),
      W_e, b_e.reshape(_EUC, 1),
      rel.reshape(1, b),
  )
  return out.reshape(b)


# speedup vs baseline: 1.0075x; 1.0075x over previous
import functools

import jax
import jax.numpy as jnp
from jax import lax
from jax.experimental import pallas as pl
from jax.experimental.pallas import tpu as pltpu
from jax.experimental.pallas import tpu_sc as plsc

_HYP, _SPH, _EUC = 16, 16, 16
_TOTAL = _HYP + _SPH + _EUC
_MAX_NORM = 1.0 - 1e-5

_NC, _NS = 2, 16
_NW = _NC * _NS


_NBUF = 16


def _sc_gather_selected(tableT, codes, b):
  pairs_per_w = b // _NW
  dmas_per_w = 2 * pairs_per_w
  groups = dmas_per_w // _NBUF

  mesh = plsc.VectorSubcoreMesh(core_axis_name="c", subcore_axis_name="s")

  @functools.partial(
      pl.kernel,
      out_type=jax.ShapeDtypeStruct((b, 32), jnp.float32),
      mesh=mesh,
      compiler_params=pltpu.CompilerParams(
          use_tc_tiling_on_sc=True, needs_layout_passes=False
      ),
      scratch_types=[
          pltpu.VMEM((dmas_per_w,), jnp.int32),
          pltpu.VMEM((_NBUF, 16, 128), jnp.float32),
          pltpu.VMEM((pairs_per_w, 32), jnp.float32),
          pltpu.SemaphoreType.DMA,
          pltpu.SemaphoreType.DMA((_NBUF,)),
      ],
  )
  def gather_kernel(tableT_hbm, codes_hbm, out_hbm, codes_v, slots, rows_v,
                    sem_c, sems):
    wid = lax.axis_index("s") * _NC + lax.axis_index("c")
    base = wid * pairs_per_w
    pltpu.async_copy(
        codes_hbm.at[pl.ds(wid * dmas_per_w, dmas_per_w)], codes_v, sem_c
    ).wait()

    iota16 = lax.iota(jnp.int32, 16)

    def lane_scalar(vec, b):
      return jnp.sum(jnp.where(iota16 == b, vec, 0))

    def fire(code, slot):
      ent = code >> 2
      off = (code & 3) << 4
      eb = pl.multiple_of((ent >> 7) << 7, 128)
      pltpu.async_copy(
          tableT_hbm.at[pl.ds(pl.multiple_of(off, 8), 8), pl.ds(eb, 128)],
          slots.at[slot, pl.ds(0, 8)],
          sems.at[slot],
      )
      pltpu.async_copy(
          tableT_hbm.at[pl.ds(pl.multiple_of(off + 8, 8), 8), pl.ds(eb, 128)],
          slots.at[slot, pl.ds(8, 8)],
          sems.at[slot],
      )

    def extract(code, j, slot):
      pltpu.make_async_copy(
          tableT_hbm.at[pl.ds(0, 16), pl.ds(0, 128)],
          slots.at[slot],
          sems.at[slot],
      ).wait()
      lane = jnp.full((16,), (code >> 2) & 127, jnp.int32)
      vals = plsc.load_gather(slots.at[slot], [iota16, lane])
      rows_v[j >> 1, pl.ds((j & 1) * 16, 16)] = vals

    cv0 = codes_v[pl.ds(0, _NBUF)]
    for bslot in range(_NBUF):
      fire(lane_scalar(cv0, bslot), bslot)

    def body(g, carry):
      jb = g * _NBUF
      cv_cur = codes_v[pl.ds(jb, _NBUF)]
      cv_nxt = codes_v[pl.ds(jb + _NBUF, _NBUF)]
      for bslot in range(_NBUF):
        extract(lane_scalar(cv_cur, bslot), jb + bslot, bslot)
        fire(lane_scalar(cv_nxt, bslot), bslot)
      return carry

    lax.fori_loop(0, groups - 1, body, 0)
    jb_last = (groups - 1) * _NBUF
    cv_last = codes_v[pl.ds(jb_last, _NBUF)]
    for bslot in range(_NBUF):
      extract(lane_scalar(cv_last, bslot), jb_last + bslot, bslot)

    pltpu.sync_copy(rows_v, out_hbm.at[pl.ds(base, pairs_per_w)])

  return gather_kernel(tableT, codes)


def _tc_body(x_ref, wh_ref, bh_ref, ws_ref, bs_ref, we_ref, be_ref,
             rel_ref, out_ref):
  x = x_ref[...]
  h = x[:16, :]
  t = x[16:, :]

  def hyp_project(v):
    n = jnp.sqrt(jnp.sum(v * v, axis=0, keepdims=True))
    f = jnp.minimum(1.0, _MAX_NORM / jnp.maximum(n, 1e-15))
    return v * f

  def sph_project(v):
    n = jnp.sqrt(jnp.sum(v * v, axis=0, keepdims=True))
    return v / jnp.maximum(n, 1e-7)

  hh = hyp_project(h)
  th = hyp_project(t)
  q = jnp.dot(wh_ref[...], hh, preferred_element_type=jnp.float32)
  q = q + bh_ref[...]
  xh = hyp_project(q)
  sq = jnp.sum((xh - th) ** 2, axis=0, keepdims=True)
  xn = jnp.sum(xh * xh, axis=0, keepdims=True)
  yn = jnp.sum(th * th, axis=0, keepdims=True)
  arg = 1.0 + 2.0 * sq / jnp.maximum((1.0 - xn) * (1.0 - yn), 1e-15)
  arg = jnp.maximum(arg, 1.0 + 1e-7)
  d_hyp = jnp.log(arg + jnp.sqrt((arg - 1.0) * (arg + 1.0)))

  hs = sph_project(h)
  ts = sph_project(t)
  s = jnp.dot(ws_ref[...], hs, preferred_element_type=jnp.float32)
  s = s + bs_ref[...]
  sn = jnp.sqrt(jnp.sum(s * s, axis=0, keepdims=True))
  shat = s / jnp.maximum(sn, 1e-7)
  dd = jnp.sum(shat * ts, axis=0, keepdims=True)
  dd = jnp.clip(dd, -1.0 + 1e-7, 1.0 - 1e-7)
  d_sph = jnp.arctan2(jnp.sqrt(1.0 - dd * dd), dd)

  e = jnp.dot(we_ref[...], h, preferred_element_type=jnp.float32)
  e = e + be_ref[...]
  d_euc = jnp.sqrt(jnp.sum((e - t) ** 2, axis=0, keepdims=True))

  rel = rel_ref[...]
  dist = jnp.where(rel == 0, d_hyp, jnp.where(rel == 1, d_sph, d_euc))
  out_ref[...] = -dist


def _tc_compute(pairsT, wh, bh, ws, bs, we, be, relT):
  b = relT.shape[1]
  blk = 2048
  grid = (b // blk,)
  small = lambda shape: pl.BlockSpec(shape, lambda i: (0,) * len(shape))
  return pl.pallas_call(
      _tc_body,
      grid=grid,
      in_specs=[
          pl.BlockSpec((32, blk), lambda i: (0, i)),
          small((_HYP, _HYP)),
          small((_HYP, 1)),
          small((_SPH, _SPH)),
          small((_SPH, 1)),
          small((_EUC, _EUC)),
          small((_EUC, 1)),
          pl.BlockSpec((1, blk), lambda i: (0, i)),
      ],
      out_specs=pl.BlockSpec((1, blk), lambda i: (0, i)),
      out_shape=jax.ShapeDtypeStruct((1, b), jnp.float32),
  )(pairsT, wh, bh, ws, bs, we, be, relT)


def kernel(entity_embeddings, W_h, b_h, W_s, b_s, W_e, b_e, heads, relations,
           tails):
  b = heads.shape[0]
  rel = relations.astype(jnp.int32)
  hc = (heads.astype(jnp.int32) << 2) | rel
  tc = (tails.astype(jnp.int32) << 2) | rel
  codes = jnp.stack([hc, tc], axis=1).reshape(2 * b)
  tableT = entity_embeddings.T

  hb = b // 2
  relT = rel.reshape(1, b)
  halves = []
  pairs1 = _sc_gather_selected(tableT, codes[: 2 * hb], hb)
  pairs2 = _sc_gather_selected(tableT, codes[2 * hb:], hb)
  for pairs, r2 in ((pairs1, relT[:, :hb]), (pairs2, relT[:, hb:])):
    halves.append(
        _tc_compute(
            pairs.T,
            W_h, b_h.reshape(_HYP, 1),
            W_s, b_s.reshape(_SPH, 1),
            W_e, b_e.reshape(_EUC, 1),
            r2,
        )
    )
  return jnp.concatenate(halves, axis=1).reshape(b)
